# baseline (speedup 1.0000x reference)
import functools

import jax
import jax.numpy as jnp
from jax import lax
from jax.experimental import pallas as pl
from jax.experimental.pallas import tpu as pltpu
from jax.experimental.pallas import tpu_sc as plsc

_B, _N, _D = 2, 2048, 768
_E = 64
_T = _B * _N
_TM = 64
_NT = 128
_NP = _NT * _TM
_NW = 32
_TPW = _T // _NW
_DA = _D + 128


def _rank_body(x_ref, rw_ref, rb_ref, gs_ref,
               xa_ref, rank_ref, e_ref, off_ref, te_ref, na_ref, run_s):
    i = pl.program_id(0)
    f32 = jnp.float32

    @pl.when(i == 0)
    def _():
        run_s[...] = jnp.zeros_like(run_s)

    @pl.when(i < 32)
    def _():
        x = x_ref[...]
        logits = jnp.dot(x, rw_ref[...], preferred_element_type=f32)
        logits = logits + rb_ref[...]
        m = jnp.max(logits, axis=-1, keepdims=True)
        s = jnp.sum(jnp.exp(logits - m), axis=-1, keepdims=True)
        w = gs_ref[0] / s
        lanes = lax.broadcasted_iota(jnp.int32, (128, _E), 1)
        e_col = jnp.min(jnp.where(logits == m, lanes, _E), axis=-1,
                        keepdims=True)
        xa_ref[...] = jnp.concatenate(
            [x, jnp.broadcast_to(w, (128, _DA - _D))], axis=1)
        e_ref[...] = e_col

        oh = (lanes == e_col).astype(f32)
        r0 = lax.broadcasted_iota(jnp.int32, (128, 128), 0)
        r1 = lax.broadcasted_iota(jnp.int32, (128, 128), 1)
        lower = (r0 >= r1).astype(f32)
        cum = jnp.dot(lower, oh, preferred_element_type=f32)
        rank_local = jnp.sum(oh * cum, axis=-1, keepdims=True) - 1.0
        run_row = run_s[0:1, :]
        rank = rank_local + jnp.sum(oh * run_row, axis=-1, keepdims=True)
        rank_ref[...] = rank.astype(jnp.int32)
        run_s[0:1, :] = run_row + jnp.sum(oh, axis=0, keepdims=True)

    @pl.when(i == 32)
    def _():
        c = run_s[0:1, :]
        p = jnp.floor((c + (_TM - 1)) * (1.0 / _TM)) * _TM
        acc = p
        for sh in (1, 2, 4, 8, 16, 32):
            z = jnp.zeros((1, sh), f32)
            acc = acc + jnp.concatenate([z, acc[:, : _E - sh]], axis=1)
        off = acc - p
        off_ref[...] = jnp.broadcast_to(off, (8, _E)).astype(jnp.int32)
        end = off + p
        total = jnp.max(end, axis=-1, keepdims=True)
        starts = lax.broadcasted_iota(jnp.int32, (_NT, 1), 0).astype(f32)
        starts = starts * float(_TM)
        te = jnp.sum((jnp.broadcast_to(end, (_NT, _E))
                      <= jnp.broadcast_to(starts, (_NT, _E))).astype(f32),
                     axis=-1, keepdims=True)
        bins = lax.broadcasted_iota(jnp.int32, (1, _E), 1).astype(f32)
        la = jnp.max(bins * (c > 0.0).astype(f32), axis=-1, keepdims=True)
        te = jnp.where(starts < total, jnp.minimum(te, float(_E - 1)), la)
        te_ref[...] = te.astype(jnp.int32)
        na_ref[...] = jnp.broadcast_to(total * (1.0 / _TM),
                                       (8, 128)).astype(jnp.int32)


def _route_rank(x_flat, router_w, router_b, gate_scale):
    return pl.pallas_call(
        _rank_body,
        grid=(33,),
        in_specs=[
            pl.BlockSpec((128, _D), lambda i: (jnp.minimum(i, 31), 0)),
            pl.BlockSpec((_D, _E), lambda i: (0, 0)),
            pl.BlockSpec((1, _E), lambda i: (0, 0)),
            pl.BlockSpec(memory_space=pltpu.SMEM),
        ],
        out_specs=[
            pl.BlockSpec((128, _DA), lambda i: (jnp.minimum(i, 31), 0)),
            pl.BlockSpec((128, 1), lambda i: (jnp.minimum(i, 31), 0)),
            pl.BlockSpec((128, 1), lambda i: (jnp.minimum(i, 31), 0)),
            pl.BlockSpec((8, _E), lambda i: (0, 0)),
            pl.BlockSpec((_NT, 1), lambda i: (0, 0)),
            pl.BlockSpec((8, 128), lambda i: (0, 0)),
        ],
        out_shape=[
            jax.ShapeDtypeStruct((_T, _DA), jnp.float32),
            jax.ShapeDtypeStruct((_T, 1), jnp.int32),
            jax.ShapeDtypeStruct((_T, 1), jnp.int32),
            jax.ShapeDtypeStruct((8, _E), jnp.int32),
            jax.ShapeDtypeStruct((_NT, 1), jnp.int32),
            jax.ShapeDtypeStruct((8, 128), jnp.int32),
        ],
        scratch_shapes=[
            pltpu.VMEM((8, _E), jnp.float32),
        ],
    )(x_flat, router_w, router_b.reshape(1, _E), gate_scale)


def _sc_scatter_rows(x_aug, e1, rank1, off1):
    mesh = plsc.VectorSubcoreMesh(core_axis_name="c", subcore_axis_name="s")

    @functools.partial(
        pl.kernel, mesh=mesh,
        out_type=(
            jax.ShapeDtypeStruct((_NP, _DA), jnp.float32),
            jax.ShapeDtypeStruct((_T,), jnp.int32),
        ),
        scratch_types=[
            pltpu.VMEM((_TPW,), jnp.int32),
            pltpu.VMEM((_TPW,), jnp.int32),
            pltpu.VMEM((_TPW,), jnp.int32),
            pltpu.VMEM((_E,), jnp.int32),
            pltpu.VMEM((_TPW, _DA), jnp.float32),
            pltpu.SemaphoreType.DMA,
        ],
    )
    def k(x_hbm, e_hbm, rank_hbm, off_hbm, out_hbm, pos_hbm,
          e_v, rank_v, pos_v, off_v, rows_v, sem):
        wid = lax.axis_index("s") * 2 + lax.axis_index("c")
        base = wid * _TPW
        pltpu.sync_copy(off_hbm, off_v)
        pltpu.sync_copy(e_hbm.at[pl.ds(base, _TPW)], e_v)
        pltpu.sync_copy(rank_hbm.at[pl.ds(base, _TPW)], rank_v)
        pltpu.sync_copy(x_hbm.at[pl.ds(base, _TPW)], rows_v)
        o_chunks = [off_v[pl.ds(k * 16, 16)] for k in range(4)]
        for g in range(_TPW // 16):
            ev = e_v[pl.ds(g * 16, 16)]
            rv = rank_v[pl.ds(g * 16, 16)]
            lo = jnp.bitwise_and(ev, 15)
            hi = jnp.right_shift(ev, 4)
            dnums = lax.GatherDimensionNumbers(
                offset_dims=(), collapsed_slice_dims=(0,),
                start_index_map=(0,))
            def _g16(chunk):
                return lax.gather(
                    chunk, lo[:, None], dnums, slice_sizes=(1,),
                    mode=lax.GatherScatterMode.PROMISE_IN_BOUNDS)
            ov = _g16(o_chunks[0])
            for kk in (1, 2, 3):
                ov = jnp.where(hi == kk, _g16(o_chunks[kk]), ov)
            pos_v[pl.ds(g * 16, 16)] = rv + ov
        pltpu.async_copy(rows_v, out_hbm.at[pos_v], sem).wait()
        pltpu.sync_copy(pos_v, pos_hbm.at[pl.ds(base, _TPW)])

    return k(x_aug, e1, rank1, off1)


def _sc_gather_rows(y_padded, pos):
    mesh = plsc.VectorSubcoreMesh(core_axis_name="c", subcore_axis_name="s")

    @functools.partial(
        pl.kernel, mesh=mesh,
        out_type=jax.ShapeDtypeStruct((_T, _D), jnp.float32),
        scratch_types=[
            pltpu.VMEM((_TPW,), jnp.int32),
            pltpu.VMEM((_TPW, _D), jnp.float32),
            pltpu.SemaphoreType.DMA,
        ],
    )
    def k(y_hbm, pos_hbm, out_hbm, idx_v, rows_v, sem):
        wid = lax.axis_index("s") * 2 + lax.axis_index("c")
        base = wid * _TPW
        pltpu.sync_copy(pos_hbm.at[pl.ds(base, _TPW)], idx_v)
        pltpu.async_copy(y_hbm.at[idx_v], rows_v, sem).wait()
        pltpu.sync_copy(rows_v, out_hbm.at[pl.ds(base, _TPW)])

    return k(y_padded, pos)


def _ffn_body(te_ref, na_ref, x_ref, w1a_ref, w1b_ref, b1_ref,
              w2a_ref, w2b_ref, b2_ref, lnw_ref, lnb_ref, o_ref):
    i = pl.program_id(0)

    @pl.when(i < na_ref[0])
    def _():
        x = x_ref[:, : _D]
        wtok = x_ref[:, _D : _D + 1]
        f32 = jnp.float32
        h = (jnp.dot(x[:, : _D // 2], w1a_ref[0, 0], preferred_element_type=f32)
             + jnp.dot(x[:, _D // 2 :], w1b_ref[0, 0], preferred_element_type=f32)
             + b1_ref[0])
        g = 0.5 * h * (1.0 + lax.erf(h * 0.7071067811865476))
        h2 = (jnp.dot(g[:, : _D // 2], w2a_ref[0, 0], preferred_element_type=f32)
              + jnp.dot(g[:, _D // 2 :], w2b_ref[0, 0], preferred_element_type=f32)
              + b2_ref[0])
        y = h2 * wtok
        mu = jnp.mean(y, axis=-1, keepdims=True)
        yc = y - mu
        var = jnp.mean(yc * yc, axis=-1, keepdims=True)
        o_ref[...] = yc * lax.rsqrt(var + 1e-5) * lnw_ref[...] + lnb_ref[...]


def _grouped_ffn(te, nact, x_padded, W1, b1, W2, b2, ln_w, ln_b):
    def _wspec(h):
        return pl.BlockSpec((1, 1, _D // 2, _D),
                            lambda i, te, na: (te[i], h, 0, 0))

    grid_spec = pltpu.PrefetchScalarGridSpec(
        num_scalar_prefetch=2,
        grid=(_NT,),
        in_specs=[
            pl.BlockSpec((_TM, _DA),
                         lambda i, te, na: (jnp.minimum(i, na[0] - 1), 0)),
            _wspec(0),
            _wspec(1),
            pl.BlockSpec((1, 1, _D), lambda i, te, na: (te[i], 0, 0)),
            _wspec(0),
            _wspec(1),
            pl.BlockSpec((1, 1, _D), lambda i, te, na: (te[i], 0, 0)),
            pl.BlockSpec((1, _D), lambda i, te, na: (0, 0)),
            pl.BlockSpec((1, _D), lambda i, te, na: (0, 0)),
        ],
        out_specs=pl.BlockSpec(
            (_TM, _D), lambda i, te, na: (jnp.minimum(i, na[0] - 1), 0)),
    )
    W1r = W1.reshape(_E, 2, _D // 2, _D)
    W2r = W2.reshape(_E, 2, _D // 2, _D)
    return pl.pallas_call(
        _ffn_body,
        grid_spec=grid_spec,
        out_shape=jax.ShapeDtypeStruct((_NP, _D), jnp.float32),
    )(te, nact, x_padded, W1r, W1r, b1.reshape(_E, 1, _D), W2r, W2r,
      b2.reshape(_E, 1, _D), ln_w.reshape(1, _D), ln_b.reshape(1, _D))


def kernel(x, router_w, router_b, W1, b1, W2, b2, gate_scale, ln_w, ln_b):
    x_flat = x.reshape(_T, _D)
    x_aug, rank_o, e_o, off8, te_o, na8 = _route_rank(
        x_flat, router_w, router_b, gate_scale)
    e1 = e_o.reshape(_T)
    rank1 = rank_o.reshape(_T)
    off1 = off8[0]
    te = te_o.reshape(_NT)
    nact = na8[0, :1]
    x_padded, pos = _sc_scatter_rows(x_aug, e1, rank1, off1)
    y_padded = _grouped_ffn(te, nact, x_padded, W1, b1, W2, b2, ln_w, ln_b)
    out_flat = _sc_gather_rows(y_padded, pos)
    return out_flat.reshape(_B, _N, _D)


# speedup vs baseline: 1.0447x; 1.0447x over previous
import functools

import jax
import jax.numpy as jnp
from jax import lax
from jax.experimental import pallas as pl
from jax.experimental.pallas import tpu as pltpu
from jax.experimental.pallas import tpu_sc as plsc

_B, _N, _D = 2, 2048, 768
_E = 64
_T = _B * _N
_TM = 64
_NT = 128
_NP = _NT * _TM
_NW = 32
_TPW = _T // _NW
_DH = _D // 2
_DA = _DH + 128


def _rank_body(x_ref, rw_ref, rb_ref, gs_ref,
               xa_ref, rank_ref, e_ref, off_ref, te_ref, na_ref, run_s):
    i = pl.program_id(0)
    f32 = jnp.float32

    @pl.when(i == 0)
    def _():
        run_s[...] = jnp.zeros_like(run_s)

    @pl.when(i < 32)
    def _():
        x = x_ref[...]
        logits = jnp.dot(x, rw_ref[...], preferred_element_type=f32)
        logits = logits + rb_ref[...]
        m = jnp.max(logits, axis=-1, keepdims=True)
        s = jnp.sum(jnp.exp(logits - m), axis=-1, keepdims=True)
        w = gs_ref[0] / s
        lanes = lax.broadcasted_iota(jnp.int32, (128, _E), 1)
        e_col = jnp.min(jnp.where(logits == m, lanes, _E), axis=-1,
                        keepdims=True)
        xb = lax.bitcast_convert_type(x, jnp.int32)
        def _bf_bits(v):
            return lax.shift_right_logical(
                v + 0x7FFF + jnp.bitwise_and(lax.shift_right_logical(v, 16), 1),
                16)
        lo = _bf_bits(xb[:, : _DH])
        hi = _bf_bits(xb[:, _DH : _D])
        wb = _bf_bits(lax.bitcast_convert_type(
            jnp.broadcast_to(w, (128, _DA - _DH)), jnp.int32))
        packed_w = jnp.bitwise_or(wb, lax.shift_left(wb, 16))
        xa_ref[...] = jnp.concatenate(
            [jnp.bitwise_or(lo, lax.shift_left(hi, 16)), packed_w], axis=1)
        e_ref[...] = e_col

        oh = (lanes == e_col).astype(f32)
        r0 = lax.broadcasted_iota(jnp.int32, (128, 128), 0)
        r1 = lax.broadcasted_iota(jnp.int32, (128, 128), 1)
        lower = (r0 >= r1).astype(f32)
        cum = jnp.dot(lower, oh, preferred_element_type=f32)
        rank_local = jnp.sum(oh * cum, axis=-1, keepdims=True) - 1.0
        run_row = run_s[0:1, :]
        rank = rank_local + jnp.sum(oh * run_row, axis=-1, keepdims=True)
        rank_ref[...] = rank.astype(jnp.int32)
        run_s[0:1, :] = run_row + jnp.sum(oh, axis=0, keepdims=True)

    @pl.when(i == 32)
    def _():
        c = run_s[0:1, :]
        p = jnp.floor((c + (_TM - 1)) * (1.0 / _TM)) * _TM
        acc = p
        for sh in (1, 2, 4, 8, 16, 32):
            z = jnp.zeros((1, sh), f32)
            acc = acc + jnp.concatenate([z, acc[:, : _E - sh]], axis=1)
        off = acc - p
        off_ref[...] = jnp.broadcast_to(off, (8, _E)).astype(jnp.int32)
        end = off + p
        total = jnp.max(end, axis=-1, keepdims=True)
        starts = lax.broadcasted_iota(jnp.int32, (_NT, 1), 0).astype(f32)
        starts = starts * float(_TM)
        te = jnp.sum((jnp.broadcast_to(end, (_NT, _E))
                      <= jnp.broadcast_to(starts, (_NT, _E))).astype(f32),
                     axis=-1, keepdims=True)
        bins = lax.broadcasted_iota(jnp.int32, (1, _E), 1).astype(f32)
        la = jnp.max(bins * (c > 0.0).astype(f32), axis=-1, keepdims=True)
        te = jnp.where(starts < total, jnp.minimum(te, float(_E - 1)), la)
        te_ref[...] = te.astype(jnp.int32)
        na_ref[...] = jnp.broadcast_to(total * (1.0 / _TM),
                                       (8, 128)).astype(jnp.int32)


def _route_rank(x_flat, router_w, router_b, gate_scale):
    return pl.pallas_call(
        _rank_body,
        grid=(33,),
        in_specs=[
            pl.BlockSpec((128, _D), lambda i: (jnp.minimum(i, 31), 0)),
            pl.BlockSpec((_D, _E), lambda i: (0, 0)),
            pl.BlockSpec((1, _E), lambda i: (0, 0)),
            pl.BlockSpec(memory_space=pltpu.SMEM),
        ],
        out_specs=[
            pl.BlockSpec((128, _DA), lambda i: (jnp.minimum(i, 31), 0)),
            pl.BlockSpec((128, 1), lambda i: (jnp.minimum(i, 31), 0)),
            pl.BlockSpec((128, 1), lambda i: (jnp.minimum(i, 31), 0)),
            pl.BlockSpec((8, _E), lambda i: (0, 0)),
            pl.BlockSpec((_NT, 1), lambda i: (0, 0)),
            pl.BlockSpec((8, 128), lambda i: (0, 0)),
        ],
        out_shape=[
            jax.ShapeDtypeStruct((_T, _DA), jnp.int32),
            jax.ShapeDtypeStruct((_T, 1), jnp.int32),
            jax.ShapeDtypeStruct((_T, 1), jnp.int32),
            jax.ShapeDtypeStruct((8, _E), jnp.int32),
            jax.ShapeDtypeStruct((_NT, 1), jnp.int32),
            jax.ShapeDtypeStruct((8, 128), jnp.int32),
        ],
        scratch_shapes=[
            pltpu.VMEM((8, _E), jnp.float32),
        ],
    )(x_flat, router_w, router_b.reshape(1, _E), gate_scale)


def _sc_scatter_rows(x_aug, e1, rank1, off1):
    mesh = plsc.VectorSubcoreMesh(core_axis_name="c", subcore_axis_name="s")

    @functools.partial(
        pl.kernel, mesh=mesh,
        out_type=(
            jax.ShapeDtypeStruct((_NP, _DA), jnp.int32),
            jax.ShapeDtypeStruct((_T,), jnp.int32),
        ),
        scratch_types=[
            pltpu.VMEM((_TPW,), jnp.int32),
            pltpu.VMEM((_TPW,), jnp.int32),
            pltpu.VMEM((_TPW,), jnp.int32),
            pltpu.VMEM((_E,), jnp.int32),
            pltpu.VMEM((_TPW, _DA), jnp.int32),
            pltpu.SemaphoreType.DMA,
        ],
    )
    def k(x_hbm, e_hbm, rank_hbm, off_hbm, out_hbm, pos_hbm,
          e_v, rank_v, pos_v, off_v, rows_v, sem):
        wid = lax.axis_index("s") * 2 + lax.axis_index("c")
        base = wid * _TPW
        pltpu.sync_copy(off_hbm, off_v)
        pltpu.sync_copy(e_hbm.at[pl.ds(base, _TPW)], e_v)
        pltpu.sync_copy(rank_hbm.at[pl.ds(base, _TPW)], rank_v)
        pltpu.sync_copy(x_hbm.at[pl.ds(base, _TPW)], rows_v)
        o_chunks = [off_v[pl.ds(k * 16, 16)] for k in range(4)]
        for g in range(_TPW // 16):
            ev = e_v[pl.ds(g * 16, 16)]
            rv = rank_v[pl.ds(g * 16, 16)]
            lo = jnp.bitwise_and(ev, 15)
            hi = jnp.right_shift(ev, 4)
            dnums = lax.GatherDimensionNumbers(
                offset_dims=(), collapsed_slice_dims=(0,),
                start_index_map=(0,))
            def _g16(chunk):
                return lax.gather(
                    chunk, lo[:, None], dnums, slice_sizes=(1,),
                    mode=lax.GatherScatterMode.PROMISE_IN_BOUNDS)
            ov = _g16(o_chunks[0])
            for kk in (1, 2, 3):
                ov = jnp.where(hi == kk, _g16(o_chunks[kk]), ov)
            pos_v[pl.ds(g * 16, 16)] = rv + ov
        pltpu.async_copy(rows_v, out_hbm.at[pos_v], sem).wait()
        pltpu.sync_copy(pos_v, pos_hbm.at[pl.ds(base, _TPW)])

    return k(x_aug, e1, rank1, off1)


def _sc_gather_rows(y_padded, pos):
    mesh = plsc.VectorSubcoreMesh(core_axis_name="c", subcore_axis_name="s")

    @functools.partial(
        pl.kernel, mesh=mesh,
        out_type=jax.ShapeDtypeStruct((_T, _D), jnp.float32),
        scratch_types=[
            pltpu.VMEM((_TPW,), jnp.int32),
            pltpu.VMEM((_TPW, _D), jnp.float32),
            pltpu.SemaphoreType.DMA,
        ],
    )
    def k(y_hbm, pos_hbm, out_hbm, idx_v, rows_v, sem):
        wid = lax.axis_index("s") * 2 + lax.axis_index("c")
        base = wid * _TPW
        pltpu.sync_copy(pos_hbm.at[pl.ds(base, _TPW)], idx_v)
        pltpu.async_copy(y_hbm.at[idx_v], rows_v, sem).wait()
        pltpu.sync_copy(rows_v, out_hbm.at[pl.ds(base, _TPW)])

    return k(y_padded, pos)


def _ffn_body(te_ref, na_ref, x_ref, w1_ref, b1_ref, w2_ref, b2_ref,
              lnw_ref, lnb_ref, o_ref):
    i = pl.program_id(0)

    @pl.when(i < na_ref[0])
    def _():
        f32 = jnp.float32
        xp = x_ref[:, : _DH]
        xe = lax.bitcast_convert_type(lax.shift_left(xp, 16), f32)
        xh = lax.bitcast_convert_type(
            jnp.bitwise_and(xp, jnp.int32(-65536)), f32)
        wtok = lax.bitcast_convert_type(
            lax.shift_left(x_ref[:, _DH : _DH + 1], 16), f32)
        w1 = w1_ref[0]
        h = (jnp.dot(xe, w1[: _DH], preferred_element_type=f32)
             + jnp.dot(xh, w1[_DH :], preferred_element_type=f32)
             + b1_ref[0])
        g = 0.5 * h * (1.0 + lax.erf(h * 0.7071067811865476))
        h2 = jnp.dot(g, w2_ref[0], preferred_element_type=jnp.float32) + b2_ref[0]
        y = h2 * wtok
        mu = jnp.mean(y, axis=-1, keepdims=True)
        yc = y - mu
        var = jnp.mean(yc * yc, axis=-1, keepdims=True)
        o_ref[...] = yc * lax.rsqrt(var + 1e-5) * lnw_ref[...] + lnb_ref[...]


def _grouped_ffn(te, nact, x_padded, W1, b1, W2, b2, ln_w, ln_b):
    grid_spec = pltpu.PrefetchScalarGridSpec(
        num_scalar_prefetch=2,
        grid=(_NT,),
        in_specs=[
            pl.BlockSpec((_TM, _DA),
                         lambda i, te, na: (jnp.minimum(i, na[0] - 1), 0)),
            pl.BlockSpec((1, _D, _D), lambda i, te, na: (te[i], 0, 0)),
            pl.BlockSpec((1, 1, _D), lambda i, te, na: (te[i], 0, 0)),
            pl.BlockSpec((1, _D, _D), lambda i, te, na: (te[i], 0, 0)),
            pl.BlockSpec((1, 1, _D), lambda i, te, na: (te[i], 0, 0)),
            pl.BlockSpec((1, _D), lambda i, te, na: (0, 0)),
            pl.BlockSpec((1, _D), lambda i, te, na: (0, 0)),
        ],
        out_specs=pl.BlockSpec(
            (_TM, _D), lambda i, te, na: (jnp.minimum(i, na[0] - 1), 0)),
    )
    return pl.pallas_call(
        _ffn_body,
        grid_spec=grid_spec,
        out_shape=jax.ShapeDtypeStruct((_NP, _D), jnp.float32),
    )(te, nact, x_padded, W1, b1.reshape(_E, 1, _D), W2,
      b2.reshape(_E, 1, _D), ln_w.reshape(1, _D), ln_b.reshape(1, _D))


def kernel(x, router_w, router_b, W1, b1, W2, b2, gate_scale, ln_w, ln_b):
    x_flat = x.reshape(_T, _D)
    x_aug, rank_o, e_o, off8, te_o, na8 = _route_rank(
        x_flat, router_w, router_b, gate_scale)
    e1 = e_o.reshape(_T)
    rank1 = rank_o.reshape(_T)
    off1 = off8[0]
    te = te_o.reshape(_NT)
    nact = na8[0, :1]
    x_padded, pos = _sc_scatter_rows(x_aug, e1, rank1, off1)
    y_padded = _grouped_ffn(te, nact, x_padded, W1, b1, W2, b2, ln_w, ln_b)
    out_flat = _sc_gather_rows(y_padded, pos)
    return out_flat.reshape(_B, _N, _D)


# speedup vs baseline: 1.0946x; 1.0478x over previous
import functools

import jax
import jax.numpy as jnp
from jax import lax
from jax.experimental import pallas as pl
from jax.experimental.pallas import tpu as pltpu
from jax.experimental.pallas import tpu_sc as plsc

_B, _N, _D = 2, 2048, 768
_E = 64
_T = _B * _N
_TM = 64
_NT = 128
_NP = _NT * _TM
_NW = 32
_TPW = _T // _NW
_DH = _D // 2
_DA = _DH + 128
_RB = 256
_NRB = _T // _RB


def _rank_body(x_ref, rw_ref, rb_ref, gs_ref,
               xa_ref, rank_ref, e_ref, off_ref, te_ref, na_ref, run_s):
    i = pl.program_id(0)
    f32 = jnp.float32

    @pl.when(i == 0)
    def _():
        run_s[...] = jnp.zeros_like(run_s)

    @pl.when(i < _NRB)
    def _():
        x = x_ref[...]
        logits = jnp.dot(x, rw_ref[...], preferred_element_type=f32)
        logits = logits + rb_ref[...]
        m = jnp.max(logits, axis=-1, keepdims=True)
        s = jnp.sum(jnp.exp(logits - m), axis=-1, keepdims=True)
        w = gs_ref[0] / s
        lanes = lax.broadcasted_iota(jnp.int32, (_RB, _E), 1)
        e_col = jnp.min(jnp.where(logits == m, lanes, _E), axis=-1,
                        keepdims=True)
        xb = lax.bitcast_convert_type(x, jnp.int32)
        def _bf_bits(v):
            return lax.shift_right_logical(
                v + 0x7FFF + jnp.bitwise_and(lax.shift_right_logical(v, 16), 1),
                16)
        lo = _bf_bits(xb[:, : _DH])
        hi = _bf_bits(xb[:, _DH : _D])
        wb = _bf_bits(lax.bitcast_convert_type(
            jnp.broadcast_to(w, (_RB, _DA - _DH)), jnp.int32))
        packed_w = jnp.bitwise_or(wb, lax.shift_left(wb, 16))
        xa_ref[...] = jnp.concatenate(
            [jnp.bitwise_or(lo, lax.shift_left(hi, 16)), packed_w], axis=1)
        e_ref[...] = e_col

        oh = (lanes == e_col).astype(f32)
        r0 = lax.broadcasted_iota(jnp.int32, (_RB, _RB), 0)
        r1 = lax.broadcasted_iota(jnp.int32, (_RB, _RB), 1)
        lower = (r0 >= r1).astype(f32)
        cum = jnp.dot(lower, oh, preferred_element_type=f32)
        rank_local = jnp.sum(oh * cum, axis=-1, keepdims=True) - 1.0
        run_row = run_s[0:1, :]
        rank = rank_local + jnp.sum(oh * run_row, axis=-1, keepdims=True)
        rank_ref[...] = rank.astype(jnp.int32)
        run_s[0:1, :] = run_row + jnp.sum(oh, axis=0, keepdims=True)

    @pl.when(i == _NRB)
    def _():
        c = run_s[0:1, :]
        p = jnp.floor((c + (_TM - 1)) * (1.0 / _TM)) * _TM
        acc = p
        for sh in (1, 2, 4, 8, 16, 32):
            z = jnp.zeros((1, sh), f32)
            acc = acc + jnp.concatenate([z, acc[:, : _E - sh]], axis=1)
        off = acc - p
        off_ref[...] = jnp.broadcast_to(off, (8, _E)).astype(jnp.int32)
        end = off + p
        total = jnp.max(end, axis=-1, keepdims=True)
        starts = lax.broadcasted_iota(jnp.int32, (_NT, 1), 0).astype(f32)
        starts = starts * float(_TM)
        te = jnp.sum((jnp.broadcast_to(end, (_NT, _E))
                      <= jnp.broadcast_to(starts, (_NT, _E))).astype(f32),
                     axis=-1, keepdims=True)
        bins = lax.broadcasted_iota(jnp.int32, (1, _E), 1).astype(f32)
        la = jnp.max(bins * (c > 0.0).astype(f32), axis=-1, keepdims=True)
        te = jnp.where(starts < total, jnp.minimum(te, float(_E - 1)), la)
        te_ref[...] = te.astype(jnp.int32)
        na_ref[...] = jnp.broadcast_to(total * (1.0 / _TM),
                                       (8, 128)).astype(jnp.int32)


def _route_rank(x_flat, router_w, router_b, gate_scale):
    return pl.pallas_call(
        _rank_body,
        grid=(_NRB + 1,),
        in_specs=[
            pl.BlockSpec((_RB, _D), lambda i: (jnp.minimum(i, _NRB - 1), 0)),
            pl.BlockSpec((_D, _E), lambda i: (0, 0)),
            pl.BlockSpec((1, _E), lambda i: (0, 0)),
            pl.BlockSpec(memory_space=pltpu.SMEM),
        ],
        out_specs=[
            pl.BlockSpec((_RB, _DA), lambda i: (jnp.minimum(i, _NRB - 1), 0)),
            pl.BlockSpec((_RB, 1), lambda i: (jnp.minimum(i, _NRB - 1), 0)),
            pl.BlockSpec((_RB, 1), lambda i: (jnp.minimum(i, _NRB - 1), 0)),
            pl.BlockSpec((8, _E), lambda i: (0, 0)),
            pl.BlockSpec((_NT, 1), lambda i: (0, 0)),
            pl.BlockSpec((8, 128), lambda i: (0, 0)),
        ],
        out_shape=[
            jax.ShapeDtypeStruct((_T, _DA), jnp.int32),
            jax.ShapeDtypeStruct((_T, 1), jnp.int32),
            jax.ShapeDtypeStruct((_T, 1), jnp.int32),
            jax.ShapeDtypeStruct((8, _E), jnp.int32),
            jax.ShapeDtypeStruct((_NT, 1), jnp.int32),
            jax.ShapeDtypeStruct((8, 128), jnp.int32),
        ],
        scratch_shapes=[
            pltpu.VMEM((8, _E), jnp.float32),
        ],
    )(x_flat, router_w, router_b.reshape(1, _E), gate_scale)


def _sc_scatter_rows(x_aug, e1, rank1, off1):
    mesh = plsc.VectorSubcoreMesh(core_axis_name="c", subcore_axis_name="s")

    @functools.partial(
        pl.kernel, mesh=mesh,
        out_type=(
            jax.ShapeDtypeStruct((_NP, _DA), jnp.int32),
            jax.ShapeDtypeStruct((_T,), jnp.int32),
        ),
        scratch_types=[
            pltpu.VMEM((_TPW,), jnp.int32),
            pltpu.VMEM((_TPW,), jnp.int32),
            pltpu.VMEM((_TPW,), jnp.int32),
            pltpu.VMEM((_E,), jnp.int32),
            pltpu.VMEM((_TPW, _DA), jnp.int32),
            pltpu.SemaphoreType.DMA,
        ],
    )
    def k(x_hbm, e_hbm, rank_hbm, off_hbm, out_hbm, pos_hbm,
          e_v, rank_v, pos_v, off_v, rows_v, sem):
        wid = lax.axis_index("s") * 2 + lax.axis_index("c")
        base = wid * _TPW
        pltpu.sync_copy(off_hbm, off_v)
        pltpu.sync_copy(e_hbm.at[pl.ds(base, _TPW)], e_v)
        pltpu.sync_copy(rank_hbm.at[pl.ds(base, _TPW)], rank_v)
        pltpu.sync_copy(x_hbm.at[pl.ds(base, _TPW)], rows_v)
        o_chunks = [off_v[pl.ds(k * 16, 16)] for k in range(4)]
        for g in range(_TPW // 16):
            ev = e_v[pl.ds(g * 16, 16)]
            rv = rank_v[pl.ds(g * 16, 16)]
            lo = jnp.bitwise_and(ev, 15)
            hi = jnp.right_shift(ev, 4)
            dnums = lax.GatherDimensionNumbers(
                offset_dims=(), collapsed_slice_dims=(0,),
                start_index_map=(0,))
            def _g16(chunk):
                return lax.gather(
                    chunk, lo[:, None], dnums, slice_sizes=(1,),
                    mode=lax.GatherScatterMode.PROMISE_IN_BOUNDS)
            ov = _g16(o_chunks[0])
            for kk in (1, 2, 3):
                ov = jnp.where(hi == kk, _g16(o_chunks[kk]), ov)
            pos_v[pl.ds(g * 16, 16)] = rv + ov
        pltpu.async_copy(rows_v, out_hbm.at[pos_v], sem).wait()
        pltpu.sync_copy(pos_v, pos_hbm.at[pl.ds(base, _TPW)])

    return k(x_aug, e1, rank1, off1)


def _sc_gather_rows(y_padded, pos):
    mesh = plsc.VectorSubcoreMesh(core_axis_name="c", subcore_axis_name="s")

    @functools.partial(
        pl.kernel, mesh=mesh,
        out_type=jax.ShapeDtypeStruct((_T, _D), jnp.float32),
        scratch_types=[
            pltpu.VMEM((_TPW,), jnp.int32),
            pltpu.VMEM((_TPW, _D), jnp.float32),
            pltpu.SemaphoreType.DMA,
        ],
    )
    def k(y_hbm, pos_hbm, out_hbm, idx_v, rows_v, sem):
        wid = lax.axis_index("s") * 2 + lax.axis_index("c")
        base = wid * _TPW
        pltpu.sync_copy(pos_hbm.at[pl.ds(base, _TPW)], idx_v)
        pltpu.async_copy(y_hbm.at[idx_v], rows_v, sem).wait()
        pltpu.sync_copy(rows_v, out_hbm.at[pl.ds(base, _TPW)])

    return k(y_padded, pos)


def _ffn_body(te_ref, na_ref, x_ref, w1_ref, b1_ref, w2_ref, b2_ref,
              lnw_ref, lnb_ref, o_ref):
    i = pl.program_id(0)

    @pl.when(i < na_ref[0])
    def _():
        f32 = jnp.float32
        xp = x_ref[:, : _DH]
        xe = lax.bitcast_convert_type(lax.shift_left(xp, 16), f32)
        xh = lax.bitcast_convert_type(
            jnp.bitwise_and(xp, jnp.int32(-65536)), f32)
        wtok = lax.bitcast_convert_type(
            lax.shift_left(x_ref[:, _DH : _DH + 1], 16), f32)
        w1 = w1_ref[0]
        h = (jnp.dot(xe, w1[: _DH], preferred_element_type=f32)
             + jnp.dot(xh, w1[_DH :], preferred_element_type=f32)
             + b1_ref[0])
        g = 0.5 * h * (1.0 + lax.erf(h * 0.7071067811865476))
        h2 = jnp.dot(g, w2_ref[0], preferred_element_type=jnp.float32) + b2_ref[0]
        y = h2 * wtok
        mu = jnp.mean(y, axis=-1, keepdims=True)
        yc = y - mu
        var = jnp.mean(yc * yc, axis=-1, keepdims=True)
        o_ref[...] = yc * lax.rsqrt(var + 1e-5) * lnw_ref[...] + lnb_ref[...]


def _grouped_ffn(te, nact, x_padded, W1, b1, W2, b2, ln_w, ln_b):
    grid_spec = pltpu.PrefetchScalarGridSpec(
        num_scalar_prefetch=2,
        grid=(_NT,),
        in_specs=[
            pl.BlockSpec((_TM, _DA),
                         lambda i, te, na: (jnp.minimum(i, na[0] - 1), 0)),
            pl.BlockSpec((1, _D, _D), lambda i, te, na: (te[i], 0, 0)),
            pl.BlockSpec((1, 1, _D), lambda i, te, na: (te[i], 0, 0)),
            pl.BlockSpec((1, _D, _D), lambda i, te, na: (te[i], 0, 0)),
            pl.BlockSpec((1, 1, _D), lambda i, te, na: (te[i], 0, 0)),
            pl.BlockSpec((1, _D), lambda i, te, na: (0, 0)),
            pl.BlockSpec((1, _D), lambda i, te, na: (0, 0)),
        ],
        out_specs=pl.BlockSpec(
            (_TM, _D), lambda i, te, na: (jnp.minimum(i, na[0] - 1), 0)),
    )
    return pl.pallas_call(
        _ffn_body,
        grid_spec=grid_spec,
        out_shape=jax.ShapeDtypeStruct((_NP, _D), jnp.float32),
    )(te, nact, x_padded, W1, b1.reshape(_E, 1, _D), W2,
      b2.reshape(_E, 1, _D), ln_w.reshape(1, _D), ln_b.reshape(1, _D))


def kernel(x, router_w, router_b, W1, b1, W2, b2, gate_scale, ln_w, ln_b):
    x_flat = x.reshape(_T, _D)
    x_aug, rank_o, e_o, off8, te_o, na8 = _route_rank(
        x_flat, router_w, router_b, gate_scale)
    e1 = e_o.reshape(_T)
    rank1 = rank_o.reshape(_T)
    off1 = off8[0]
    te = te_o.reshape(_NT)
    nact = na8[0, :1]
    x_padded, pos = _sc_scatter_rows(x_aug, e1, rank1, off1)
    y_padded = _grouped_ffn(te, nact, x_padded, W1, b1, W2, b2, ln_w, ln_b)
    out_flat = _sc_gather_rows(y_padded, pos)
    return out_flat.reshape(_B, _N, _D)


# speedup vs baseline: 1.1234x; 1.0263x over previous
import functools

import jax
import jax.numpy as jnp
from jax import lax
from jax.experimental import pallas as pl
from jax.experimental.pallas import tpu as pltpu
from jax.experimental.pallas import tpu_sc as plsc

_B, _N, _D = 2, 2048, 768
_E = 64
_T = _B * _N
_TM = 64
_NT = 128
_NP = _NT * _TM
_NW = 32
_TPW = _T // _NW
_DH = _D // 2
_DA = _DH + 128
_RB = 512
_NRB = _T // _RB


def _rank_body(x_ref, rw_ref, rb_ref, gs_ref,
               xa_ref, rank_ref, e_ref, off_ref, te_ref, na_ref, run_s):
    i = pl.program_id(0)
    f32 = jnp.float32

    @pl.when(i == 0)
    def _():
        run_s[...] = jnp.zeros_like(run_s)

    @pl.when(i < _NRB)
    def _():
        x = x_ref[...]
        logits = jnp.dot(x, rw_ref[...], preferred_element_type=f32)
        logits = logits + rb_ref[...]
        m = jnp.max(logits, axis=-1, keepdims=True)
        s = jnp.sum(jnp.exp(logits - m), axis=-1, keepdims=True)
        w = gs_ref[0] / s
        lanes = lax.broadcasted_iota(jnp.int32, (_RB, _E), 1)
        e_col = jnp.min(jnp.where(logits == m, lanes, _E), axis=-1,
                        keepdims=True)
        xb = lax.bitcast_convert_type(x, jnp.int32)
        def _bf_bits(v):
            return lax.shift_right_logical(
                v + 0x7FFF + jnp.bitwise_and(lax.shift_right_logical(v, 16), 1),
                16)
        lo = _bf_bits(xb[:, : _DH])
        hi = _bf_bits(xb[:, _DH : _D])
        wb = _bf_bits(lax.bitcast_convert_type(
            jnp.broadcast_to(w, (_RB, _DA - _DH)), jnp.int32))
        packed_w = jnp.bitwise_or(wb, lax.shift_left(wb, 16))
        xa_ref[...] = jnp.concatenate(
            [jnp.bitwise_or(lo, lax.shift_left(hi, 16)), packed_w], axis=1)
        e_ref[...] = e_col

        oh = (lanes == e_col).astype(f32)
        r0 = lax.broadcasted_iota(jnp.int32, (_RB, _RB), 0)
        r1 = lax.broadcasted_iota(jnp.int32, (_RB, _RB), 1)
        lower = (r0 >= r1).astype(f32)
        cum = jnp.dot(lower, oh, preferred_element_type=f32)
        rank_local = jnp.sum(oh * cum, axis=-1, keepdims=True) - 1.0
        run_row = run_s[0:1, :]
        rank = rank_local + jnp.sum(oh * run_row, axis=-1, keepdims=True)
        rank_ref[...] = rank.astype(jnp.int32)
        run_s[0:1, :] = run_row + jnp.sum(oh, axis=0, keepdims=True)

    @pl.when(i == _NRB)
    def _():
        c = run_s[0:1, :]
        p = jnp.floor((c + (_TM - 1)) * (1.0 / _TM)) * _TM
        acc = p
        for sh in (1, 2, 4, 8, 16, 32):
            z = jnp.zeros((1, sh), f32)
            acc = acc + jnp.concatenate([z, acc[:, : _E - sh]], axis=1)
        off = acc - p
        off_ref[...] = jnp.broadcast_to(off, (8, _E)).astype(jnp.int32)
        end = off + p
        total = jnp.max(end, axis=-1, keepdims=True)
        starts = lax.broadcasted_iota(jnp.int32, (_NT, 1), 0).astype(f32)
        starts = starts * float(_TM)
        te = jnp.sum((jnp.broadcast_to(end, (_NT, _E))
                      <= jnp.broadcast_to(starts, (_NT, _E))).astype(f32),
                     axis=-1, keepdims=True)
        bins = lax.broadcasted_iota(jnp.int32, (1, _E), 1).astype(f32)
        la = jnp.max(bins * (c > 0.0).astype(f32), axis=-1, keepdims=True)
        te = jnp.where(starts < total, jnp.minimum(te, float(_E - 1)), la)
        te_ref[...] = te.astype(jnp.int32)
        na_ref[...] = jnp.broadcast_to(total * (1.0 / _TM),
                                       (8, 128)).astype(jnp.int32)


def _route_rank(x_flat, router_w, router_b, gate_scale):
    return pl.pallas_call(
        _rank_body,
        grid=(_NRB + 1,),
        in_specs=[
            pl.BlockSpec((_RB, _D), lambda i: (jnp.minimum(i, _NRB - 1), 0)),
            pl.BlockSpec((_D, _E), lambda i: (0, 0)),
            pl.BlockSpec((1, _E), lambda i: (0, 0)),
            pl.BlockSpec(memory_space=pltpu.SMEM),
        ],
        out_specs=[
            pl.BlockSpec((_RB, _DA), lambda i: (jnp.minimum(i, _NRB - 1), 0)),
            pl.BlockSpec((_RB, 1), lambda i: (jnp.minimum(i, _NRB - 1), 0)),
            pl.BlockSpec((_RB, 1), lambda i: (jnp.minimum(i, _NRB - 1), 0)),
            pl.BlockSpec((8, _E), lambda i: (0, 0)),
            pl.BlockSpec((_NT, 1), lambda i: (0, 0)),
            pl.BlockSpec((8, 128), lambda i: (0, 0)),
        ],
        out_shape=[
            jax.ShapeDtypeStruct((_T, _DA), jnp.int32),
            jax.ShapeDtypeStruct((_T, 1), jnp.int32),
            jax.ShapeDtypeStruct((_T, 1), jnp.int32),
            jax.ShapeDtypeStruct((8, _E), jnp.int32),
            jax.ShapeDtypeStruct((_NT, 1), jnp.int32),
            jax.ShapeDtypeStruct((8, 128), jnp.int32),
        ],
        scratch_shapes=[
            pltpu.VMEM((8, _E), jnp.float32),
        ],
    )(x_flat, router_w, router_b.reshape(1, _E), gate_scale)


def _sc_scatter_rows(x_aug, e1, rank1, off1):
    mesh = plsc.VectorSubcoreMesh(core_axis_name="c", subcore_axis_name="s")

    @functools.partial(
        pl.kernel, mesh=mesh,
        out_type=(
            jax.ShapeDtypeStruct((_NP, _DA), jnp.int32),
            jax.ShapeDtypeStruct((_T,), jnp.int32),
        ),
        scratch_types=[
            pltpu.VMEM((_TPW,), jnp.int32),
            pltpu.VMEM((_TPW,), jnp.int32),
            pltpu.VMEM((_TPW,), jnp.int32),
            pltpu.VMEM((_E,), jnp.int32),
            pltpu.VMEM((_TPW, _DA), jnp.int32),
            pltpu.SemaphoreType.DMA,
        ],
    )
    def k(x_hbm, e_hbm, rank_hbm, off_hbm, out_hbm, pos_hbm,
          e_v, rank_v, pos_v, off_v, rows_v, sem):
        wid = lax.axis_index("s") * 2 + lax.axis_index("c")
        base = wid * _TPW
        pltpu.sync_copy(off_hbm, off_v)
        pltpu.sync_copy(e_hbm.at[pl.ds(base, _TPW)], e_v)
        pltpu.sync_copy(rank_hbm.at[pl.ds(base, _TPW)], rank_v)
        pltpu.sync_copy(x_hbm.at[pl.ds(base, _TPW)], rows_v)
        o_chunks = [off_v[pl.ds(k * 16, 16)] for k in range(4)]
        for g in range(_TPW // 16):
            ev = e_v[pl.ds(g * 16, 16)]
            rv = rank_v[pl.ds(g * 16, 16)]
            lo = jnp.bitwise_and(ev, 15)
            hi = jnp.right_shift(ev, 4)
            dnums = lax.GatherDimensionNumbers(
                offset_dims=(), collapsed_slice_dims=(0,),
                start_index_map=(0,))
            def _g16(chunk):
                return lax.gather(
                    chunk, lo[:, None], dnums, slice_sizes=(1,),
                    mode=lax.GatherScatterMode.PROMISE_IN_BOUNDS)
            ov = _g16(o_chunks[0])
            for kk in (1, 2, 3):
                ov = jnp.where(hi == kk, _g16(o_chunks[kk]), ov)
            pos_v[pl.ds(g * 16, 16)] = rv + ov
        pltpu.async_copy(rows_v, out_hbm.at[pos_v], sem).wait()
        pltpu.sync_copy(pos_v, pos_hbm.at[pl.ds(base, _TPW)])

    return k(x_aug, e1, rank1, off1)


def _sc_gather_rows(y_padded, pos):
    mesh = plsc.VectorSubcoreMesh(core_axis_name="c", subcore_axis_name="s")

    @functools.partial(
        pl.kernel, mesh=mesh,
        out_type=jax.ShapeDtypeStruct((_T, _D), jnp.float32),
        scratch_types=[
            pltpu.VMEM((_TPW,), jnp.int32),
            pltpu.VMEM((_TPW, _D), jnp.float32),
            pltpu.SemaphoreType.DMA,
        ],
    )
    def k(y_hbm, pos_hbm, out_hbm, idx_v, rows_v, sem):
        wid = lax.axis_index("s") * 2 + lax.axis_index("c")
        base = wid * _TPW
        pltpu.sync_copy(pos_hbm.at[pl.ds(base, _TPW)], idx_v)
        pltpu.async_copy(y_hbm.at[idx_v], rows_v, sem).wait()
        pltpu.sync_copy(rows_v, out_hbm.at[pl.ds(base, _TPW)])

    return k(y_padded, pos)


def _ffn_body(te_ref, na_ref, x_ref, w1_ref, b1_ref, w2_ref, b2_ref,
              lnw_ref, lnb_ref, o_ref):
    i = pl.program_id(0)

    @pl.when(i < na_ref[0])
    def _():
        f32 = jnp.float32
        xp = x_ref[:, : _DH]
        xe = lax.bitcast_convert_type(lax.shift_left(xp, 16), f32)
        xh = lax.bitcast_convert_type(
            jnp.bitwise_and(xp, jnp.int32(-65536)), f32)
        wtok = lax.bitcast_convert_type(
            lax.shift_left(x_ref[:, _DH : _DH + 1], 16), f32)
        w1 = w1_ref[0]
        h = (jnp.dot(xe, w1[: _DH], preferred_element_type=f32)
             + jnp.dot(xh, w1[_DH :], preferred_element_type=f32)
             + b1_ref[0])
        g = 0.5 * h * (1.0 + lax.erf(h * 0.7071067811865476))
        h2 = jnp.dot(g, w2_ref[0], preferred_element_type=jnp.float32) + b2_ref[0]
        y = h2 * wtok
        mu = jnp.mean(y, axis=-1, keepdims=True)
        yc = y - mu
        var = jnp.mean(yc * yc, axis=-1, keepdims=True)
        o_ref[...] = yc * lax.rsqrt(var + 1e-5) * lnw_ref[...] + lnb_ref[...]


def _grouped_ffn(te, nact, x_padded, W1, b1, W2, b2, ln_w, ln_b):
    grid_spec = pltpu.PrefetchScalarGridSpec(
        num_scalar_prefetch=2,
        grid=(_NT,),
        in_specs=[
            pl.BlockSpec((_TM, _DA),
                         lambda i, te, na: (jnp.minimum(i, na[0] - 1), 0)),
            pl.BlockSpec((1, _D, _D), lambda i, te, na: (te[i], 0, 0)),
            pl.BlockSpec((1, 1, _D), lambda i, te, na: (te[i], 0, 0)),
            pl.BlockSpec((1, _D, _D), lambda i, te, na: (te[i], 0, 0)),
            pl.BlockSpec((1, 1, _D), lambda i, te, na: (te[i], 0, 0)),
            pl.BlockSpec((1, _D), lambda i, te, na: (0, 0)),
            pl.BlockSpec((1, _D), lambda i, te, na: (0, 0)),
        ],
        out_specs=pl.BlockSpec(
            (_TM, _D), lambda i, te, na: (jnp.minimum(i, na[0] - 1), 0)),
    )
    return pl.pallas_call(
        _ffn_body,
        grid_spec=grid_spec,
        out_shape=jax.ShapeDtypeStruct((_NP, _D), jnp.float32),
    )(te, nact, x_padded, W1, b1.reshape(_E, 1, _D), W2,
      b2.reshape(_E, 1, _D), ln_w.reshape(1, _D), ln_b.reshape(1, _D))


def kernel(x, router_w, router_b, W1, b1, W2, b2, gate_scale, ln_w, ln_b):
    x_flat = x.reshape(_T, _D)
    x_aug, rank_o, e_o, off8, te_o, na8 = _route_rank(
        x_flat, router_w, router_b, gate_scale)
    e1 = e_o.reshape(_T)
    rank1 = rank_o.reshape(_T)
    off1 = off8[0]
    te = te_o.reshape(_NT)
    nact = na8[0, :1]
    x_padded, pos = _sc_scatter_rows(x_aug, e1, rank1, off1)
    y_padded = _grouped_ffn(te, nact, x_padded, W1, b1, W2, b2, ln_w, ln_b)
    out_flat = _sc_gather_rows(y_padded, pos)
    return out_flat.reshape(_B, _N, _D)


# speedup vs baseline: 1.1315x; 1.0072x over previous
import functools

import jax
import jax.numpy as jnp
from jax import lax
from jax.experimental import pallas as pl
from jax.experimental.pallas import tpu as pltpu
from jax.experimental.pallas import tpu_sc as plsc

_B, _N, _D = 2, 2048, 768
_E = 64
_T = _B * _N
_TM = 64
_NT = 128
_NP = _NT * _TM
_NW = 32
_TPW = _T // _NW
_DH = _D // 2
_DA = _DH + 128
_RB = 1024
_NRB = _T // _RB


def _rank_body(x_ref, rw_ref, rb_ref, gs_ref,
               xa_ref, rank_ref, e_ref, off_ref, te_ref, na_ref, run_s):
    i = pl.program_id(0)
    f32 = jnp.float32

    @pl.when(i == 0)
    def _():
        run_s[...] = jnp.zeros_like(run_s)

    @pl.when(i < _NRB)
    def _():
        x = x_ref[...]
        logits = jnp.dot(x, rw_ref[...], preferred_element_type=f32)
        logits = logits + rb_ref[...]
        m = jnp.max(logits, axis=-1, keepdims=True)
        s = jnp.sum(jnp.exp(logits - m), axis=-1, keepdims=True)
        w = gs_ref[0] / s
        lanes = lax.broadcasted_iota(jnp.int32, (_RB, _E), 1)
        e_col = jnp.min(jnp.where(logits == m, lanes, _E), axis=-1,
                        keepdims=True)
        xb = lax.bitcast_convert_type(x, jnp.int32)
        def _bf_bits(v):
            return lax.shift_right_logical(
                v + 0x7FFF + jnp.bitwise_and(lax.shift_right_logical(v, 16), 1),
                16)
        lo = _bf_bits(xb[:, : _DH])
        hi = _bf_bits(xb[:, _DH : _D])
        wb = _bf_bits(lax.bitcast_convert_type(
            jnp.broadcast_to(w, (_RB, _DA - _DH)), jnp.int32))
        packed_w = jnp.bitwise_or(wb, lax.shift_left(wb, 16))
        xa_ref[...] = jnp.concatenate(
            [jnp.bitwise_or(lo, lax.shift_left(hi, 16)), packed_w], axis=1)
        e_ref[...] = e_col

        oh = (lanes == e_col).astype(f32)
        r0 = lax.broadcasted_iota(jnp.int32, (_RB, _RB), 0)
        r1 = lax.broadcasted_iota(jnp.int32, (_RB, _RB), 1)
        lower = (r0 >= r1).astype(f32)
        cum = jnp.dot(lower, oh, preferred_element_type=f32)
        rank_local = jnp.sum(oh * cum, axis=-1, keepdims=True) - 1.0
        run_row = run_s[0:1, :]
        rank = rank_local + jnp.sum(oh * run_row, axis=-1, keepdims=True)
        rank_ref[...] = rank.astype(jnp.int32)
        run_s[0:1, :] = run_row + jnp.sum(oh, axis=0, keepdims=True)

    @pl.when(i == _NRB)
    def _():
        c = run_s[0:1, :]
        p = jnp.floor((c + (_TM - 1)) * (1.0 / _TM)) * _TM
        acc = p
        for sh in (1, 2, 4, 8, 16, 32):
            z = jnp.zeros((1, sh), f32)
            acc = acc + jnp.concatenate([z, acc[:, : _E - sh]], axis=1)
        off = acc - p
        off_ref[...] = jnp.broadcast_to(off, (8, _E)).astype(jnp.int32)
        end = off + p
        total = jnp.max(end, axis=-1, keepdims=True)
        starts = lax.broadcasted_iota(jnp.int32, (_NT, 1), 0).astype(f32)
        starts = starts * float(_TM)
        te = jnp.sum((jnp.broadcast_to(end, (_NT, _E))
                      <= jnp.broadcast_to(starts, (_NT, _E))).astype(f32),
                     axis=-1, keepdims=True)
        bins = lax.broadcasted_iota(jnp.int32, (1, _E), 1).astype(f32)
        la = jnp.max(bins * (c > 0.0).astype(f32), axis=-1, keepdims=True)
        te = jnp.where(starts < total, jnp.minimum(te, float(_E - 1)), la)
        te_ref[...] = te.astype(jnp.int32)
        na_ref[...] = jnp.broadcast_to(total * (1.0 / _TM),
                                       (8, 128)).astype(jnp.int32)


def _route_rank(x_flat, router_w, router_b, gate_scale):
    return pl.pallas_call(
        _rank_body,
        grid=(_NRB + 1,),
        in_specs=[
            pl.BlockSpec((_RB, _D), lambda i: (jnp.minimum(i, _NRB - 1), 0)),
            pl.BlockSpec((_D, _E), lambda i: (0, 0)),
            pl.BlockSpec((1, _E), lambda i: (0, 0)),
            pl.BlockSpec(memory_space=pltpu.SMEM),
        ],
        out_specs=[
            pl.BlockSpec((_RB, _DA), lambda i: (jnp.minimum(i, _NRB - 1), 0)),
            pl.BlockSpec((_RB, 1), lambda i: (jnp.minimum(i, _NRB - 1), 0)),
            pl.BlockSpec((_RB, 1), lambda i: (jnp.minimum(i, _NRB - 1), 0)),
            pl.BlockSpec((8, _E), lambda i: (0, 0)),
            pl.BlockSpec((_NT, 1), lambda i: (0, 0)),
            pl.BlockSpec((8, 128), lambda i: (0, 0)),
        ],
        out_shape=[
            jax.ShapeDtypeStruct((_T, _DA), jnp.int32),
            jax.ShapeDtypeStruct((_T, 1), jnp.int32),
            jax.ShapeDtypeStruct((_T, 1), jnp.int32),
            jax.ShapeDtypeStruct((8, _E), jnp.int32),
            jax.ShapeDtypeStruct((_NT, 1), jnp.int32),
            jax.ShapeDtypeStruct((8, 128), jnp.int32),
        ],
        scratch_shapes=[
            pltpu.VMEM((8, _E), jnp.float32),
        ],
    )(x_flat, router_w, router_b.reshape(1, _E), gate_scale)


def _sc_scatter_rows(x_aug, e1, rank1, off1):
    mesh = plsc.VectorSubcoreMesh(core_axis_name="c", subcore_axis_name="s")

    @functools.partial(
        pl.kernel, mesh=mesh,
        out_type=(
            jax.ShapeDtypeStruct((_NP, _DA), jnp.int32),
            jax.ShapeDtypeStruct((_T,), jnp.int32),
        ),
        scratch_types=[
            pltpu.VMEM((_TPW,), jnp.int32),
            pltpu.VMEM((_TPW,), jnp.int32),
            pltpu.VMEM((_TPW,), jnp.int32),
            pltpu.VMEM((_E,), jnp.int32),
            pltpu.VMEM((_TPW, _DA), jnp.int32),
            pltpu.SemaphoreType.DMA,
        ],
    )
    def k(x_hbm, e_hbm, rank_hbm, off_hbm, out_hbm, pos_hbm,
          e_v, rank_v, pos_v, off_v, rows_v, sem):
        wid = lax.axis_index("s") * 2 + lax.axis_index("c")
        base = wid * _TPW
        pltpu.sync_copy(off_hbm, off_v)
        pltpu.sync_copy(e_hbm.at[pl.ds(base, _TPW)], e_v)
        pltpu.sync_copy(rank_hbm.at[pl.ds(base, _TPW)], rank_v)
        pltpu.sync_copy(x_hbm.at[pl.ds(base, _TPW)], rows_v)
        o_chunks = [off_v[pl.ds(k * 16, 16)] for k in range(4)]
        for g in range(_TPW // 16):
            ev = e_v[pl.ds(g * 16, 16)]
            rv = rank_v[pl.ds(g * 16, 16)]
            lo = jnp.bitwise_and(ev, 15)
            hi = jnp.right_shift(ev, 4)
            dnums = lax.GatherDimensionNumbers(
                offset_dims=(), collapsed_slice_dims=(0,),
                start_index_map=(0,))
            def _g16(chunk):
                return lax.gather(
                    chunk, lo[:, None], dnums, slice_sizes=(1,),
                    mode=lax.GatherScatterMode.PROMISE_IN_BOUNDS)
            ov = _g16(o_chunks[0])
            for kk in (1, 2, 3):
                ov = jnp.where(hi == kk, _g16(o_chunks[kk]), ov)
            pos_v[pl.ds(g * 16, 16)] = rv + ov
        pltpu.async_copy(rows_v, out_hbm.at[pos_v], sem).wait()
        pltpu.sync_copy(pos_v, pos_hbm.at[pl.ds(base, _TPW)])

    return k(x_aug, e1, rank1, off1)


def _sc_gather_rows(y_padded, pos):
    mesh = plsc.VectorSubcoreMesh(core_axis_name="c", subcore_axis_name="s")

    @functools.partial(
        pl.kernel, mesh=mesh,
        out_type=jax.ShapeDtypeStruct((_T, _D), jnp.float32),
        scratch_types=[
            pltpu.VMEM((_TPW,), jnp.int32),
            pltpu.VMEM((_TPW, _D), jnp.float32),
            pltpu.SemaphoreType.DMA,
        ],
    )
    def k(y_hbm, pos_hbm, out_hbm, idx_v, rows_v, sem):
        wid = lax.axis_index("s") * 2 + lax.axis_index("c")
        base = wid * _TPW
        pltpu.sync_copy(pos_hbm.at[pl.ds(base, _TPW)], idx_v)
        pltpu.async_copy(y_hbm.at[idx_v], rows_v, sem).wait()
        pltpu.sync_copy(rows_v, out_hbm.at[pl.ds(base, _TPW)])

    return k(y_padded, pos)


def _ffn_body(te_ref, na_ref, x_ref, w1_ref, b1_ref, w2_ref, b2_ref,
              lnw_ref, lnb_ref, o_ref):
    i = pl.program_id(0)

    @pl.when(i < na_ref[0])
    def _():
        f32 = jnp.float32
        xp = x_ref[:, : _DH]
        xe = lax.bitcast_convert_type(lax.shift_left(xp, 16), f32)
        xh = lax.bitcast_convert_type(
            jnp.bitwise_and(xp, jnp.int32(-65536)), f32)
        wtok = lax.bitcast_convert_type(
            lax.shift_left(x_ref[:, _DH : _DH + 1], 16), f32)
        w1 = w1_ref[0]
        h = (jnp.dot(xe, w1[: _DH], preferred_element_type=f32)
             + jnp.dot(xh, w1[_DH :], preferred_element_type=f32)
             + b1_ref[0])
        g = 0.5 * h * (1.0 + lax.erf(h * 0.7071067811865476))
        h2 = jnp.dot(g, w2_ref[0], preferred_element_type=jnp.float32) + b2_ref[0]
        y = h2 * wtok
        mu = jnp.mean(y, axis=-1, keepdims=True)
        yc = y - mu
        var = jnp.mean(yc * yc, axis=-1, keepdims=True)
        o_ref[...] = yc * lax.rsqrt(var + 1e-5) * lnw_ref[...] + lnb_ref[...]


def _grouped_ffn(te, nact, x_padded, W1, b1, W2, b2, ln_w, ln_b):
    grid_spec = pltpu.PrefetchScalarGridSpec(
        num_scalar_prefetch=2,
        grid=(_NT,),
        in_specs=[
            pl.BlockSpec((_TM, _DA),
                         lambda i, te, na: (jnp.minimum(i, na[0] - 1), 0)),
            pl.BlockSpec((1, _D, _D), lambda i, te, na: (te[i], 0, 0)),
            pl.BlockSpec((1, 1, _D), lambda i, te, na: (te[i], 0, 0)),
            pl.BlockSpec((1, _D, _D), lambda i, te, na: (te[i], 0, 0)),
            pl.BlockSpec((1, 1, _D), lambda i, te, na: (te[i], 0, 0)),
            pl.BlockSpec((1, _D), lambda i, te, na: (0, 0)),
            pl.BlockSpec((1, _D), lambda i, te, na: (0, 0)),
        ],
        out_specs=pl.BlockSpec(
            (_TM, _D), lambda i, te, na: (jnp.minimum(i, na[0] - 1), 0)),
    )
    return pl.pallas_call(
        _ffn_body,
        grid_spec=grid_spec,
        out_shape=jax.ShapeDtypeStruct((_NP, _D), jnp.float32),
    )(te, nact, x_padded, W1, b1.reshape(_E, 1, _D), W2,
      b2.reshape(_E, 1, _D), ln_w.reshape(1, _D), ln_b.reshape(1, _D))


def kernel(x, router_w, router_b, W1, b1, W2, b2, gate_scale, ln_w, ln_b):
    x_flat = x.reshape(_T, _D)
    x_aug, rank_o, e_o, off8, te_o, na8 = _route_rank(
        x_flat, router_w, router_b, gate_scale)
    e1 = e_o.reshape(_T)
    rank1 = rank_o.reshape(_T)
    off1 = off8[0]
    te = te_o.reshape(_NT)
    nact = na8[0, :1]
    x_padded, pos = _sc_scatter_rows(x_aug, e1, rank1, off1)
    y_padded = _grouped_ffn(te, nact, x_padded, W1, b1, W2, b2, ln_w, ln_b)
    out_flat = _sc_gather_rows(y_padded, pos)
    return out_flat.reshape(_B, _N, _D)


# speedup vs baseline: 1.1556x; 1.0213x over previous
import functools

import jax
import jax.numpy as jnp
from jax import lax
from jax.experimental import pallas as pl
from jax.experimental.pallas import tpu as pltpu
from jax.experimental.pallas import tpu_sc as plsc

_B, _N, _D = 2, 2048, 768
_E = 64
_T = _B * _N
_TM = 64
_NT = 128
_NP = _NT * _TM
_NW = 32
_TPW = _T // _NW
_DH = _D // 2
_DA = _DH + 128
_RB = 1024
_NRB = _T // _RB


def _rank_body(x_ref, rw_ref, rb_ref, gs_ref,
               xa_ref, er_ref, off_ref, te_ref, na_ref, run_s):
    i = pl.program_id(0)
    f32 = jnp.float32

    @pl.when(i == 0)
    def _():
        run_s[...] = jnp.zeros_like(run_s)

    @pl.when(i < _NRB)
    def _():
        x = x_ref[...]
        logits = jnp.dot(x, rw_ref[...], preferred_element_type=f32)
        logits = logits + rb_ref[...]
        m = jnp.max(logits, axis=-1, keepdims=True)
        s = jnp.sum(jnp.exp(logits - m), axis=-1, keepdims=True)
        w = gs_ref[0] / s
        lanes = lax.broadcasted_iota(jnp.int32, (_RB, _E), 1)
        e_col = jnp.min(jnp.where(logits == m, lanes, _E), axis=-1,
                        keepdims=True)
        xb = lax.bitcast_convert_type(x, jnp.int32)
        def _bf_bits(v):
            return lax.shift_right_logical(
                v + 0x7FFF + jnp.bitwise_and(lax.shift_right_logical(v, 16), 1),
                16)
        lo = _bf_bits(xb[:, : _DH])
        hi = _bf_bits(xb[:, _DH : _D])
        wb = _bf_bits(lax.bitcast_convert_type(
            jnp.broadcast_to(w, (_RB, _DA - _DH)), jnp.int32))
        packed_w = jnp.bitwise_or(wb, lax.shift_left(wb, 16))
        xa_ref[...] = jnp.concatenate(
            [jnp.bitwise_or(lo, lax.shift_left(hi, 16)), packed_w], axis=1)

        oh = (lanes == e_col).astype(f32)
        r0 = lax.broadcasted_iota(jnp.int32, (_RB, _RB), 0)
        r1 = lax.broadcasted_iota(jnp.int32, (_RB, _RB), 1)
        lower = (r0 >= r1).astype(f32)
        cum = jnp.dot(lower, oh, preferred_element_type=f32)
        rank_local = jnp.sum(oh * cum, axis=-1, keepdims=True) - 1.0
        run_row = run_s[0:1, :]
        rank = rank_local + jnp.sum(oh * run_row, axis=-1, keepdims=True)
        er_ref[...] = jnp.bitwise_or(lax.shift_left(e_col, 12),
                                     rank.astype(jnp.int32))
        run_s[0:1, :] = run_row + jnp.sum(oh, axis=0, keepdims=True)

    @pl.when(i == _NRB)
    def _():
        c = run_s[0:1, :]
        p = jnp.floor((c + (_TM - 1)) * (1.0 / _TM)) * _TM
        acc = p
        for sh in (1, 2, 4, 8, 16, 32):
            z = jnp.zeros((1, sh), f32)
            acc = acc + jnp.concatenate([z, acc[:, : _E - sh]], axis=1)
        off = acc - p
        off_ref[...] = jnp.broadcast_to(off, (8, _E)).astype(jnp.int32)
        end = off + p
        total = jnp.max(end, axis=-1, keepdims=True)
        starts = lax.broadcasted_iota(jnp.int32, (_NT, 1), 0).astype(f32)
        starts = starts * float(_TM)
        te = jnp.sum((jnp.broadcast_to(end, (_NT, _E))
                      <= jnp.broadcast_to(starts, (_NT, _E))).astype(f32),
                     axis=-1, keepdims=True)
        bins = lax.broadcasted_iota(jnp.int32, (1, _E), 1).astype(f32)
        la = jnp.max(bins * (c > 0.0).astype(f32), axis=-1, keepdims=True)
        te = jnp.where(starts < total, jnp.minimum(te, float(_E - 1)), la)
        te_ref[...] = te.astype(jnp.int32)
        na_ref[...] = jnp.broadcast_to(total * (1.0 / _TM),
                                       (8, 128)).astype(jnp.int32)


def _route_rank(x_flat, router_w, router_b, gate_scale):
    return pl.pallas_call(
        _rank_body,
        grid=(_NRB + 1,),
        in_specs=[
            pl.BlockSpec((_RB, _D), lambda i: (jnp.minimum(i, _NRB - 1), 0)),
            pl.BlockSpec((_D, _E), lambda i: (0, 0)),
            pl.BlockSpec((1, _E), lambda i: (0, 0)),
            pl.BlockSpec(memory_space=pltpu.SMEM),
        ],
        out_specs=[
            pl.BlockSpec((_RB, _DA), lambda i: (jnp.minimum(i, _NRB - 1), 0)),
            pl.BlockSpec((_RB, 1), lambda i: (jnp.minimum(i, _NRB - 1), 0)),
            pl.BlockSpec((8, _E), lambda i: (0, 0)),
            pl.BlockSpec((_NT, 1), lambda i: (0, 0)),
            pl.BlockSpec((8, 128), lambda i: (0, 0)),
        ],
        out_shape=[
            jax.ShapeDtypeStruct((_T, _DA), jnp.int32),
            jax.ShapeDtypeStruct((_T, 1), jnp.int32),
            jax.ShapeDtypeStruct((8, _E), jnp.int32),
            jax.ShapeDtypeStruct((_NT, 1), jnp.int32),
            jax.ShapeDtypeStruct((8, 128), jnp.int32),
        ],
        scratch_shapes=[
            pltpu.VMEM((8, _E), jnp.float32),
        ],
    )(x_flat, router_w, router_b.reshape(1, _E), gate_scale)


def _sc_scatter_rows(x_aug, er1, off8):
    mesh = plsc.VectorSubcoreMesh(core_axis_name="c", subcore_axis_name="s")

    @functools.partial(
        pl.kernel, mesh=mesh,
        out_type=(
            jax.ShapeDtypeStruct((_NP, _DA), jnp.int32),
            jax.ShapeDtypeStruct((_T,), jnp.int32),
        ),
        scratch_types=[
            pltpu.VMEM((_TPW,), jnp.int32),
            pltpu.VMEM((_TPW,), jnp.int32),
            pltpu.VMEM((_E,), jnp.int32),
            pltpu.VMEM((_TPW, _DA), jnp.int32),
            pltpu.SemaphoreType.DMA,
        ],
    )
    def k(x_hbm, er_hbm, off_hbm, out_hbm, pos_hbm,
          er_v, pos_v, off_v, rows_v, sem):
        wid = lax.axis_index("s") * 2 + lax.axis_index("c")
        base = wid * _TPW
        pltpu.sync_copy(off_hbm.at[0], off_v)
        pltpu.sync_copy(er_hbm.at[pl.ds(base, _TPW)], er_v)
        pltpu.sync_copy(x_hbm.at[pl.ds(base, _TPW)], rows_v)
        o_chunks = [off_v[pl.ds(k * 16, 16)] for k in range(4)]
        for g in range(_TPW // 16):
            erv = er_v[pl.ds(g * 16, 16)]
            ev = lax.shift_right_logical(erv, 12)
            rv = jnp.bitwise_and(erv, 4095)
            lo = jnp.bitwise_and(ev, 15)
            hi = jnp.right_shift(ev, 4)
            dnums = lax.GatherDimensionNumbers(
                offset_dims=(), collapsed_slice_dims=(0,),
                start_index_map=(0,))
            def _g16(chunk):
                return lax.gather(
                    chunk, lo[:, None], dnums, slice_sizes=(1,),
                    mode=lax.GatherScatterMode.PROMISE_IN_BOUNDS)
            ov = _g16(o_chunks[0])
            for kk in (1, 2, 3):
                ov = jnp.where(hi == kk, _g16(o_chunks[kk]), ov)
            pos_v[pl.ds(g * 16, 16)] = rv + ov
        pltpu.async_copy(rows_v, out_hbm.at[pos_v], sem).wait()
        pltpu.sync_copy(pos_v, pos_hbm.at[pl.ds(base, _TPW)])

    return k(x_aug, er1, off8)


def _sc_gather_rows(y_padded, pos):
    mesh = plsc.VectorSubcoreMesh(core_axis_name="c", subcore_axis_name="s")

    @functools.partial(
        pl.kernel, mesh=mesh,
        out_type=jax.ShapeDtypeStruct((_T, _D), jnp.float32),
        scratch_types=[
            pltpu.VMEM((_TPW,), jnp.int32),
            pltpu.VMEM((_TPW, _D), jnp.float32),
            pltpu.SemaphoreType.DMA,
        ],
    )
    def k(y_hbm, pos_hbm, out_hbm, idx_v, rows_v, sem):
        wid = lax.axis_index("s") * 2 + lax.axis_index("c")
        base = wid * _TPW
        pltpu.sync_copy(pos_hbm.at[pl.ds(base, _TPW)], idx_v)
        pltpu.async_copy(y_hbm.at[idx_v], rows_v, sem).wait()
        pltpu.sync_copy(rows_v, out_hbm.at[pl.ds(base, _TPW)])

    return k(y_padded, pos)


def _ffn_body(te_ref, na_ref, x_ref, w1_ref, b1_ref, w2_ref, b2_ref,
              lnw_ref, lnb_ref, o_ref):
    i = pl.program_id(0)

    @pl.when(i < na_ref[0, 0])
    def _():
        f32 = jnp.float32
        xp = x_ref[:, : _DH]
        xe = lax.bitcast_convert_type(lax.shift_left(xp, 16), f32)
        xh = lax.bitcast_convert_type(
            jnp.bitwise_and(xp, jnp.int32(-65536)), f32)
        wtok = lax.bitcast_convert_type(
            lax.shift_left(x_ref[:, _DH : _DH + 1], 16), f32)
        w1 = w1_ref[0]
        h = (jnp.dot(xe, w1[: _DH], preferred_element_type=f32)
             + jnp.dot(xh, w1[_DH :], preferred_element_type=f32)
             + b1_ref[0])
        g = 0.5 * h * (1.0 + lax.erf(h * 0.7071067811865476))
        h2 = jnp.dot(g, w2_ref[0], preferred_element_type=jnp.float32) + b2_ref[0]
        y = h2 * wtok
        mu = jnp.mean(y, axis=-1, keepdims=True)
        yc = y - mu
        var = jnp.mean(yc * yc, axis=-1, keepdims=True)
        o_ref[...] = yc * lax.rsqrt(var + 1e-5) * lnw_ref[...] + lnb_ref[...]


def _grouped_ffn(te, nact, x_padded, W1, b1, W2, b2, ln_w, ln_b):
    grid_spec = pltpu.PrefetchScalarGridSpec(
        num_scalar_prefetch=2,
        grid=(_NT,),
        in_specs=[
            pl.BlockSpec((_TM, _DA),
                         lambda i, te, na: (jnp.minimum(i, na[0, 0] - 1), 0)),
            pl.BlockSpec((1, _D, _D), lambda i, te, na: (te[i, 0], 0, 0)),
            pl.BlockSpec((1, 1, _D), lambda i, te, na: (te[i, 0], 0, 0)),
            pl.BlockSpec((1, _D, _D), lambda i, te, na: (te[i, 0], 0, 0)),
            pl.BlockSpec((1, 1, _D), lambda i, te, na: (te[i, 0], 0, 0)),
            pl.BlockSpec((1, _D), lambda i, te, na: (0, 0)),
            pl.BlockSpec((1, _D), lambda i, te, na: (0, 0)),
        ],
        out_specs=pl.BlockSpec(
            (_TM, _D), lambda i, te, na: (jnp.minimum(i, na[0, 0] - 1), 0)),
    )
    return pl.pallas_call(
        _ffn_body,
        grid_spec=grid_spec,
        out_shape=jax.ShapeDtypeStruct((_NP, _D), jnp.float32),
    )(te, nact, x_padded, W1, b1.reshape(_E, 1, _D), W2,
      b2.reshape(_E, 1, _D), ln_w.reshape(1, _D), ln_b.reshape(1, _D))


def kernel(x, router_w, router_b, W1, b1, W2, b2, gate_scale, ln_w, ln_b):
    x_flat = x.reshape(_T, _D)
    x_aug, er_o, off8, te_o, na8 = _route_rank(
        x_flat, router_w, router_b, gate_scale)
    x_padded, pos = _sc_scatter_rows(x_aug, er_o.reshape(_T), off8)
    y_padded = _grouped_ffn(te_o, na8, x_padded, W1, b1, W2, b2, ln_w, ln_b)
    out_flat = _sc_gather_rows(y_padded, pos)
    return out_flat.reshape(_B, _N, _D)
